# Initial kernel scaffold; baseline (speedup 1.0000x reference)
import functools

import jax
import jax.numpy as jnp
from jax import lax
from jax.experimental import pallas as pl
from jax.experimental.pallas import tpu as pltpu
from jax.experimental.pallas import tpu_sc as plsc

N = 10000
E = 320000
F = 128
NC, NS = 2, 16
EPS = E // (NC * NS)
EB = 128
NFULL = EPS // EB
TAIL = EPS - NFULL * EB
RPT = N // NS
ZROWS = 125

_mesh = plsc.VectorSubcoreMesh(core_axis_name="c", subcore_axis_name="s")


@functools.partial(
    pl.kernel,
    out_type=jax.ShapeDtypeStruct((NC, N, F), jnp.float32),
    mesh=_mesh,
    scratch_types=[
        pltpu.VMEM((EB,), jnp.int32),
        pltpu.VMEM((EB,), jnp.int32),
        pltpu.VMEM((EB, F), jnp.float32),
        pltpu.VMEM((TAIL,), jnp.int32),
        pltpu.VMEM((TAIL,), jnp.int32),
        pltpu.VMEM((TAIL, F), jnp.float32),
        pltpu.VMEM((ZROWS, F), jnp.float32),
        pltpu.VMEM_SHARED((N, F), jnp.float32),
        pltpu.SemaphoreType.DMA,
    ],
)
def _sc_agg(x_hbm, src_hbm, dst_hbm, out_hbm,
            sidx, didx, rows, sidx_t, didx_t, rows_t, zbuf, acc, sem):
    cid = lax.axis_index("c")
    sid = lax.axis_index("s")

    zvec = jnp.zeros((16,), jnp.float32)

    def _zrow(r, carry):
        for c in range(F // 16):
            zbuf[r, pl.ds(c * 16, 16)] = zvec
        return carry

    lax.fori_loop(0, ZROWS, _zrow, 0)
    row0 = sid * RPT
    for k in range(RPT // ZROWS):
        pltpu.sync_copy(zbuf, acc.at[pl.ds(row0 + k * ZROWS, ZROWS)])
    plsc.subcore_barrier()

    ebase = (cid * NS + sid) * EPS

    def _ebody(g, carry):
        off = ebase + g * EB
        pltpu.sync_copy(src_hbm.at[pl.ds(off, EB)], sidx)
        pltpu.sync_copy(dst_hbm.at[pl.ds(off, EB)], didx)
        pltpu.async_copy(x_hbm.at[sidx], rows, sem).wait()
        pltpu.sync_copy(rows, acc.at[didx], add=True)
        return carry

    lax.fori_loop(0, NFULL, _ebody, 0)

    toff = ebase + NFULL * EB
    pltpu.sync_copy(src_hbm.at[pl.ds(toff, TAIL)], sidx_t)
    pltpu.sync_copy(dst_hbm.at[pl.ds(toff, TAIL)], didx_t)
    pltpu.async_copy(x_hbm.at[sidx_t], rows_t, sem).wait()
    pltpu.sync_copy(rows_t, acc.at[didx_t], add=True)

    plsc.subcore_barrier()

    for k in range(RPT // ZROWS):
        r = row0 + k * ZROWS
        pltpu.sync_copy(acc.at[pl.ds(r, ZROWS)], out_hbm.at[cid, pl.ds(r, ZROWS)])


def _tc_layer_call(ya, yb, x, W, Ws, b, *, final):
    M, Fin = x.shape
    Fo = W.shape[1]
    BM = 1000

    def body(ya_ref, yb_ref, x_ref, W_ref, Ws_ref, b_ref, o_ref):
        y = ya_ref[...] + yb_ref[...]
        acc = jnp.dot(y, W_ref[...], preferred_element_type=jnp.float32)
        acc += jnp.dot(x_ref[...], Ws_ref[...], preferred_element_type=jnp.float32)
        logits = acc + b_ref[...]
        if final:
            m = jnp.max(logits, axis=-1, keepdims=True)
            z = logits - m
            lse = jnp.log(jnp.sum(jnp.exp(z), axis=-1, keepdims=True))
            o_ref[...] = z - lse
        else:
            o_ref[...] = logits

    return pl.pallas_call(
        body,
        grid=(M // BM,),
        in_specs=[
            pl.BlockSpec((BM, Fin), lambda i: (i, 0)),
            pl.BlockSpec((BM, Fin), lambda i: (i, 0)),
            pl.BlockSpec((BM, Fin), lambda i: (i, 0)),
            pl.BlockSpec((Fin, Fo), lambda i: (0, 0)),
            pl.BlockSpec((Fin, Fo), lambda i: (0, 0)),
            pl.BlockSpec((1, Fo), lambda i: (0, 0)),
        ],
        out_specs=pl.BlockSpec((BM, Fo), lambda i: (i, 0)),
        out_shape=jax.ShapeDtypeStruct((M, Fo), jnp.float32),
    )(ya, yb, x, W, Ws, b.reshape(1, Fo))


def kernel(fea, edge_index, W_in, Ws_in, b_in, W_mid, Ws_mid, b_mid,
           W_out, Ws_out, b_out):
    src = edge_index[0]
    dst = edge_index[1]
    y = _sc_agg(fea, src, dst)
    x1 = _tc_layer_call(y[0], y[1], fea, W_in, Ws_in, b_in, final=False)
    y = _sc_agg(x1, src, dst)
    x2 = _tc_layer_call(y[0], y[1], x1, W_mid, Ws_mid, b_mid, final=False)
    y = _sc_agg(x2, src, dst)
    return _tc_layer_call(y[0], y[1], x2, W_out, Ws_out, b_out, final=True)


# speedup vs baseline: 5.5330x; 5.5330x over previous
import functools

import jax
import jax.numpy as jnp
from jax import lax
from jax.experimental import pallas as pl
from jax.experimental.pallas import tpu as pltpu
from jax.experimental.pallas import tpu_sc as plsc

N = 10000
E = 320000
F = 128
NC, NS = 2, 16
EPS = E // (NC * NS)
EB = 128
NFULL = EPS // EB
TAIL = EPS - NFULL * EB
CHUNK = 128
NCH = N // CHUNK
CTAIL = N - NCH * CHUNK

_mesh = plsc.VectorSubcoreMesh(core_axis_name="c", subcore_axis_name="s")


@functools.partial(
    pl.kernel,
    out_type=jax.ShapeDtypeStruct((NC, N, F), jnp.float32),
    mesh=_mesh,
    scratch_types=[
        pltpu.VMEM((EB,), jnp.int32),
        pltpu.VMEM((EB,), jnp.int32),
        pltpu.VMEM((EB, F), jnp.float32),
        pltpu.VMEM((TAIL,), jnp.int32),
        pltpu.VMEM((TAIL,), jnp.int32),
        pltpu.VMEM((TAIL, F), jnp.float32),
        pltpu.VMEM((CHUNK, F), jnp.float32),
        pltpu.VMEM_SHARED((N, F), jnp.float32),
        pltpu.SemaphoreType.DMA,
    ],
)
def _sc_agg(x_hbm, src_hbm, dst_hbm, out_hbm,
            sidx, didx, rows, sidx_t, didx_t, rows_t, zbuf, acc, sem):
    cid = lax.axis_index("c")
    sid = lax.axis_index("s")

    zvec = jnp.zeros((16,), jnp.float32)

    def _zrow(r, carry):
        for c in range(F // 16):
            zbuf[r, pl.ds(c * 16, 16)] = zvec
        return carry

    lax.fori_loop(0, CHUNK, _zrow, 0)

    def _zchunk(c, carry):
        @pl.when(c % NS == sid)
        def _():
            pltpu.sync_copy(zbuf, acc.at[pl.ds(c * CHUNK, CHUNK)])
        return carry

    lax.fori_loop(0, NCH, _zchunk, 0)

    @pl.when(sid == NS - 1)
    def _ztail():
        pltpu.sync_copy(zbuf.at[pl.ds(0, CTAIL)],
                        acc.at[pl.ds(NCH * CHUNK, CTAIL)])

    plsc.subcore_barrier()

    ebase = (cid * NS + sid) * EPS

    def _ebody(g, carry):
        off = ebase + g * EB
        pltpu.sync_copy(src_hbm.at[pl.ds(off, EB)], sidx)
        pltpu.sync_copy(dst_hbm.at[pl.ds(off, EB)], didx)
        pltpu.async_copy(x_hbm.at[sidx], rows, sem).wait()
        pltpu.sync_copy(rows, acc.at[didx], add=True)
        return carry

    lax.fori_loop(0, NFULL, _ebody, 0)

    toff = ebase + NFULL * EB
    pltpu.sync_copy(src_hbm.at[pl.ds(toff, TAIL)], sidx_t)
    pltpu.sync_copy(dst_hbm.at[pl.ds(toff, TAIL)], didx_t)
    pltpu.async_copy(x_hbm.at[sidx_t], rows_t, sem).wait()
    pltpu.sync_copy(rows_t, acc.at[didx_t], add=True)

    plsc.subcore_barrier()

    def _ochunk(c, carry):
        @pl.when(c % NS == sid)
        def _():
            pltpu.sync_copy(acc.at[pl.ds(c * CHUNK, CHUNK)],
                            out_hbm.at[cid, pl.ds(c * CHUNK, CHUNK)])
        return carry

    lax.fori_loop(0, NCH, _ochunk, 0)

    @pl.when(sid == NS - 1)
    def _otail():
        pltpu.sync_copy(acc.at[pl.ds(NCH * CHUNK, CTAIL)],
                        out_hbm.at[cid, pl.ds(NCH * CHUNK, CTAIL)])


def _tc_layer_call(ya, yb, x, W, Ws, b, *, final):
    M, Fin = x.shape
    Fo = W.shape[1]
    BM = 1000

    def body(ya_ref, yb_ref, x_ref, W_ref, Ws_ref, b_ref, o_ref):
        y = ya_ref[...] + yb_ref[...]
        acc = jnp.dot(y, W_ref[...], preferred_element_type=jnp.float32)
        acc += jnp.dot(x_ref[...], Ws_ref[...], preferred_element_type=jnp.float32)
        logits = acc + b_ref[...]
        if final:
            m = jnp.max(logits, axis=-1, keepdims=True)
            z = logits - m
            lse = jnp.log(jnp.sum(jnp.exp(z), axis=-1, keepdims=True))
            o_ref[...] = z - lse
        else:
            o_ref[...] = logits

    return pl.pallas_call(
        body,
        grid=(M // BM,),
        in_specs=[
            pl.BlockSpec((BM, Fin), lambda i: (i, 0)),
            pl.BlockSpec((BM, Fin), lambda i: (i, 0)),
            pl.BlockSpec((BM, Fin), lambda i: (i, 0)),
            pl.BlockSpec((Fin, Fo), lambda i: (0, 0)),
            pl.BlockSpec((Fin, Fo), lambda i: (0, 0)),
            pl.BlockSpec((1, Fo), lambda i: (0, 0)),
        ],
        out_specs=pl.BlockSpec((BM, Fo), lambda i: (i, 0)),
        out_shape=jax.ShapeDtypeStruct((M, Fo), jnp.float32),
    )(ya, yb, x, W, Ws, b.reshape(1, Fo))


def kernel(fea, edge_index, W_in, Ws_in, b_in, W_mid, Ws_mid, b_mid,
           W_out, Ws_out, b_out):
    src = edge_index[0]
    dst = edge_index[1]
    y = _sc_agg(fea, src, dst)
    x1 = _tc_layer_call(y[0], y[1], fea, W_in, Ws_in, b_in, final=False)
    y = _sc_agg(x1, src, dst)
    x2 = _tc_layer_call(y[0], y[1], x1, W_mid, Ws_mid, b_mid, final=False)
    y = _sc_agg(x2, src, dst)
    return _tc_layer_call(y[0], y[1], x2, W_out, Ws_out, b_out, final=True)
